# Initial kernel scaffold; baseline (speedup 1.0000x reference)
import jax
import jax.numpy as jnp
from jax.experimental import pallas as pl


def kernel(features, edge_index, W1, b1, W2, b2):
    raise NotImplementedError("write your pallas kernel here")


# speedup vs baseline: 6.5805x; 6.5805x over previous
import functools

import jax
import jax.numpy as jnp
from jax import lax
from jax.experimental import pallas as pl
from jax.experimental.pallas import tpu as pltpu
from jax.experimental.pallas import tpu_sc as plsc

N = 50000
E = 1600000
F = 1433
H = 16
O = 7

NC = 2
NS = 16
NW = NC * NS
CH = 80
ROWS = 20480
EPAD = ROWS * CH
WROWS = ROWS // NW
IBLK = 32
NOUTER = WROWS // IBLK
SINK = N

NPAD = 51200
DSTRIPE = NPAD // NS
APAD = N + 48
ASTRIPE = APAD // NS

_MESH = plsc.VectorSubcoreMesh(core_axis_name="c", subcore_axis_name="s")


def _deg_body(src2d, dst2d, out, acc_s, acc_d, zero_v, ones_v, sblk, dblk):
    c = lax.axis_index("c")
    s = lax.axis_index("s")
    w = c * NS + s

    zeros16 = jnp.zeros((16,), jnp.float32)

    def z_body(i, carry):
        zero_v[pl.ds(i * 16, 16)] = zeros16
        return carry

    lax.fori_loop(0, DSTRIPE // 16, z_body, None)
    for i in range(CH // 16):
        ones_v[pl.ds(i * 16, 16)] = jnp.ones((16,), jnp.float32)

    pltpu.sync_copy(zero_v, acc_s.at[pl.ds(s * DSTRIPE, DSTRIPE)])
    pltpu.sync_copy(zero_v, acc_d.at[pl.ds(s * DSTRIPE, DSTRIPE)])
    plsc.subcore_barrier()

    row0 = w * WROWS

    def outer(o, carry):
        pltpu.sync_copy(src2d.at[pl.ds(row0 + o * IBLK, IBLK), :], sblk)
        pltpu.sync_copy(dst2d.at[pl.ds(row0 + o * IBLK, IBLK), :], dblk)

        def inner(j, icarry):
            pltpu.sync_copy(ones_v, acc_s.at[sblk.at[j]], add=True)
            pltpu.sync_copy(ones_v, acc_d.at[dblk.at[j]], add=True)
            return icarry

        lax.fori_loop(0, IBLK, inner, None)
        return carry

    lax.fori_loop(0, NOUTER, outer, None)
    plsc.subcore_barrier()

    pltpu.sync_copy(acc_s.at[pl.ds(s * DSTRIPE, DSTRIPE)],
                    out.at[c, 0, pl.ds(s * DSTRIPE, DSTRIPE)])
    pltpu.sync_copy(acc_d.at[pl.ds(s * DSTRIPE, DSTRIPE)],
                    out.at[c, 1, pl.ds(s * DSTRIPE, DSTRIPE)])


_deg = pl.kernel(
    _deg_body,
    out_type=jax.ShapeDtypeStruct((NC, 2, NPAD), jnp.float32),
    mesh=_MESH,
    compiler_params=pltpu.CompilerParams(use_tc_tiling_on_sc=False),
    scratch_types=[
        pltpu.VMEM_SHARED((NPAD,), jnp.float32),
        pltpu.VMEM_SHARED((NPAD,), jnp.float32),
        pltpu.VMEM((DSTRIPE,), jnp.float32),
        pltpu.VMEM((CH,), jnp.float32),
        pltpu.VMEM((IBLK, CH), jnp.int32),
        pltpu.VMEM((IBLK, CH), jnp.int32),
    ],
)


def _agg_body(hs, src2d, dst2d, out, acc, zero_v, sblk, dblk, rows_v):
    c = lax.axis_index("c")
    s = lax.axis_index("s")
    w = c * NS + s

    zeros16 = jnp.zeros((16,), jnp.float32)

    def z_body(i, carry):
        zero_v[i, :] = zeros16
        return carry

    lax.fori_loop(0, ASTRIPE, z_body, None)
    pltpu.sync_copy(zero_v, acc.at[pl.ds(s * ASTRIPE, ASTRIPE), :])
    plsc.subcore_barrier()

    row0 = w * WROWS

    def outer(o, carry):
        pltpu.sync_copy(src2d.at[pl.ds(row0 + o * IBLK, IBLK), :], sblk)
        pltpu.sync_copy(dst2d.at[pl.ds(row0 + o * IBLK, IBLK), :], dblk)

        def inner(j, icarry):
            pltpu.sync_copy(hs.at[sblk.at[j]], rows_v)
            pltpu.sync_copy(rows_v, acc.at[dblk.at[j]], add=True)
            return icarry

        lax.fori_loop(0, IBLK, inner, None)
        return carry

    lax.fori_loop(0, NOUTER, outer, None)
    plsc.subcore_barrier()

    pltpu.sync_copy(acc.at[pl.ds(s * ASTRIPE, ASTRIPE), :],
                    out.at[c, pl.ds(s * ASTRIPE, ASTRIPE), :])


_agg = pl.kernel(
    _agg_body,
    out_type=jax.ShapeDtypeStruct((NC, APAD, H), jnp.float32),
    mesh=_MESH,
    compiler_params=pltpu.CompilerParams(use_tc_tiling_on_sc=False),
    scratch_types=[
        pltpu.VMEM_SHARED((APAD, H), jnp.float32),
        pltpu.VMEM((ASTRIPE, H), jnp.float32),
        pltpu.VMEM((IBLK, CH), jnp.int32),
        pltpu.VMEM((IBLK, CH), jnp.int32),
        pltpu.VMEM((CH, H), jnp.float32),
    ],
)


_BM = 1000


def _mm1_body(x_ref, w_ref, o_ref):
    o_ref[...] = lax.dot_general(
        x_ref[...], w_ref[...], (((1,), (0,)), ((), ())),
        precision=lax.Precision.HIGHEST, preferred_element_type=jnp.float32)


_mm1 = pl.pallas_call(
    _mm1_body,
    grid=(N // _BM,),
    in_specs=[
        pl.BlockSpec((_BM, F), lambda i: (i, 0)),
        pl.BlockSpec((F, H), lambda i: (0, 0)),
    ],
    out_specs=pl.BlockSpec((_BM, H), lambda i: (i, 0)),
    out_shape=jax.ShapeDtypeStruct((N, H), jnp.float32),
)


_BN = 2000


def _scale_body(degp_ref, h1_ref, h1s_ref, dsi_ref, dso_ref):
    d = degp_ref[...]
    dso = lax.rsqrt(jnp.maximum(d[:, 0:1] + d[:, 2:3], 1.0))
    dsi = lax.rsqrt(jnp.maximum(d[:, 1:2] + d[:, 3:4], 1.0))
    h1s_ref[...] = h1_ref[...] * dso
    dsi_ref[...] = dsi
    dso_ref[...] = dso


_scale = pl.pallas_call(
    _scale_body,
    grid=(N // _BN,),
    in_specs=[
        pl.BlockSpec((_BN, 4), lambda i: (i, 0)),
        pl.BlockSpec((_BN, H), lambda i: (i, 0)),
    ],
    out_specs=[
        pl.BlockSpec((_BN, H), lambda i: (i, 0)),
        pl.BlockSpec((_BN, 1), lambda i: (i, 0)),
        pl.BlockSpec((_BN, 1), lambda i: (i, 0)),
    ],
    out_shape=[
        jax.ShapeDtypeStruct((N, H), jnp.float32),
        jax.ShapeDtypeStruct((N, 1), jnp.float32),
        jax.ShapeDtypeStruct((N, 1), jnp.float32),
    ],
)


def _post1_body(p_ref, dsi_ref, dso_ref, b1_ref, g_ref):
    q = (p_ref[0] + p_ref[1]) * dsi_ref[...]
    g_ref[...] = jnp.maximum(q + b1_ref[...], 0.0) * dso_ref[...]


_post1 = pl.pallas_call(
    _post1_body,
    grid=(N // _BN,),
    in_specs=[
        pl.BlockSpec((NC, _BN, H), lambda i: (0, i, 0)),
        pl.BlockSpec((_BN, 1), lambda i: (i, 0)),
        pl.BlockSpec((_BN, 1), lambda i: (i, 0)),
        pl.BlockSpec((1, H), lambda i: (0, 0)),
    ],
    out_specs=pl.BlockSpec((_BN, H), lambda i: (i, 0)),
    out_shape=jax.ShapeDtypeStruct((N, H), jnp.float32),
)


def _post2_body(p_ref, dsi_ref, w2_ref, b2_ref, o_ref):
    q = (p_ref[0] + p_ref[1]) * dsi_ref[...]
    o_ref[...] = lax.dot_general(
        q, w2_ref[...], (((1,), (0,)), ((), ())),
        precision=lax.Precision.HIGHEST,
        preferred_element_type=jnp.float32) + b2_ref[...]


_post2 = pl.pallas_call(
    _post2_body,
    grid=(N // _BN,),
    in_specs=[
        pl.BlockSpec((NC, _BN, H), lambda i: (0, i, 0)),
        pl.BlockSpec((_BN, 1), lambda i: (i, 0)),
        pl.BlockSpec((H, O), lambda i: (0, 0)),
        pl.BlockSpec((1, O), lambda i: (0, 0)),
    ],
    out_specs=pl.BlockSpec((_BN, O), lambda i: (i, 0)),
    out_shape=jax.ShapeDtypeStruct((N, O), jnp.float32),
)


def kernel(features, edge_index, W1, b1, W2, b2):
    pad = jnp.full((EPAD - E,), SINK, jnp.int32)
    src2d = jnp.concatenate([edge_index[0], pad]).reshape(ROWS, CH)
    dst2d = jnp.concatenate([edge_index[1], pad]).reshape(ROWS, CH)

    degp = _deg(src2d, dst2d)
    degp4 = degp[:, :, :N].reshape(4, N).T
    h1 = _mm1(features, W1)
    h1s, dsi, dso = _scale(degp4, h1)
    h1s = jnp.pad(h1s, ((0, APAD - N), (0, 0)))
    p1 = _agg(h1s, src2d, dst2d)[:, :N]
    g = _post1(p1, dsi, dso, b1.reshape(1, H))
    g = jnp.pad(g, ((0, APAD - N), (0, 0)))
    p2 = _agg(g, src2d, dst2d)[:, :N]
    out = _post2(p2, dsi, W2, b2.reshape(1, O))
    return out


# speedup vs baseline: 12.0733x; 1.8347x over previous
import functools

import jax
import jax.numpy as jnp
from jax import lax
from jax.experimental import pallas as pl
from jax.experimental.pallas import tpu as pltpu
from jax.experimental.pallas import tpu_sc as plsc

N = 50000
E = 1600000
F = 1433
H = 16
O = 7

NC = 2
NS = 16
NW = NC * NS
CH = 128
ROWS = 12544
EPAD = ROWS * CH
WROWS = ROWS // NW
IBLK = 28
NOUTER = WROWS // IBLK
D = 4
SINK = N

NPAD = 51200
DSTRIPE = NPAD // NS
APAD = N + 48
ASTRIPE = APAD // NS

_MESH = plsc.VectorSubcoreMesh(core_axis_name="c", subcore_axis_name="s")


def _deg_body(ei, out, acc_s, acc_d, zero_v, ones_v, iblk):
    c = lax.axis_index("c")
    s = lax.axis_index("s")
    w = c * NS + s

    zeros16 = jnp.zeros((16,), jnp.float32)

    def z_body(i, carry):
        zero_v[pl.ds(i * 16, 16)] = zeros16
        return carry

    lax.fori_loop(0, DSTRIPE // 16, z_body, None)
    for i in range(CH // 16):
        ones_v[pl.ds(i * 16, 16)] = jnp.ones((16,), jnp.float32)

    pltpu.sync_copy(zero_v, acc_s.at[pl.ds(s * DSTRIPE, DSTRIPE)])
    pltpu.sync_copy(zero_v, acc_d.at[pl.ds(s * DSTRIPE, DSTRIPE)])
    plsc.subcore_barrier()

    row0 = w * WROWS

    def outer(o, carry):
        pltpu.sync_copy(ei.at[pl.ds(row0 + o * IBLK, IBLK)], iblk)

        def inner(j, icarry):
            pltpu.sync_copy(ones_v, acc_s.at[iblk.at[j, 0]], add=True)
            pltpu.sync_copy(ones_v, acc_d.at[iblk.at[j, 1]], add=True)
            return icarry

        lax.fori_loop(0, IBLK, inner, None)
        return carry

    lax.fori_loop(0, NOUTER, outer, None)
    plsc.subcore_barrier()

    pltpu.sync_copy(acc_s.at[pl.ds(s * DSTRIPE, DSTRIPE)],
                    out.at[c, 0, pl.ds(s * DSTRIPE, DSTRIPE)])
    pltpu.sync_copy(acc_d.at[pl.ds(s * DSTRIPE, DSTRIPE)],
                    out.at[c, 1, pl.ds(s * DSTRIPE, DSTRIPE)])


_deg = pl.kernel(
    _deg_body,
    out_type=jax.ShapeDtypeStruct((NC, 2, NPAD), jnp.float32),
    mesh=_MESH,
    compiler_params=pltpu.CompilerParams(use_tc_tiling_on_sc=False),
    scratch_types=[
        pltpu.VMEM_SHARED((NPAD,), jnp.float32),
        pltpu.VMEM_SHARED((NPAD,), jnp.float32),
        pltpu.VMEM((DSTRIPE,), jnp.float32),
        pltpu.VMEM((CH,), jnp.float32),
        pltpu.VMEM((IBLK, 2, CH), jnp.int32),
    ],
)


def _agg_body(hs, ei, out, acc, zero_v, iblk, rows_v, gsem):
    c = lax.axis_index("c")
    s = lax.axis_index("s")
    w = c * NS + s

    zeros16 = jnp.zeros((16,), jnp.float32)

    def z_body(i, carry):
        zero_v[i, :] = zeros16
        return carry

    lax.fori_loop(0, ASTRIPE, z_body, None)
    pltpu.sync_copy(zero_v, acc.at[pl.ds(s * ASTRIPE, ASTRIPE), :])
    plsc.subcore_barrier()

    row0 = w * WROWS

    def outer(o, carry):
        pltpu.sync_copy(ei.at[pl.ds(row0 + o * IBLK, IBLK)], iblk)
        for k in range(D):
            pltpu.async_copy(hs.at[iblk.at[k, 0]], rows_v.at[k], gsem.at[k])

        def inner(j, icarry):
            b = lax.rem(j, D)
            pltpu.make_async_copy(hs.at[iblk.at[j, 0]], rows_v.at[b],
                                  gsem.at[b]).wait()
            pltpu.sync_copy(rows_v.at[b], acc.at[iblk.at[j, 1]], add=True)

            @pl.when(j < IBLK - D)
            def _():
                pltpu.async_copy(hs.at[iblk.at[j + D, 0]], rows_v.at[b],
                                 gsem.at[b])

            return icarry

        lax.fori_loop(0, IBLK, inner, None)
        return carry

    lax.fori_loop(0, NOUTER, outer, None)
    plsc.subcore_barrier()

    pltpu.sync_copy(acc.at[pl.ds(s * ASTRIPE, ASTRIPE), :],
                    out.at[c, pl.ds(s * ASTRIPE, ASTRIPE), :])


_agg = pl.kernel(
    _agg_body,
    out_type=jax.ShapeDtypeStruct((NC, APAD, H), jnp.float32),
    mesh=_MESH,
    compiler_params=pltpu.CompilerParams(use_tc_tiling_on_sc=False),
    scratch_types=[
        pltpu.VMEM_SHARED((APAD, H), jnp.float32),
        pltpu.VMEM((ASTRIPE, H), jnp.float32),
        pltpu.VMEM((IBLK, 2, CH), jnp.int32),
        pltpu.VMEM((D, CH, H), jnp.float32),
        pltpu.SemaphoreType.DMA((D,)),
    ],
)


_BM = 1000


def _mm1_body(x_ref, w_ref, o_ref):
    o_ref[...] = lax.dot_general(
        x_ref[...], w_ref[...], (((1,), (0,)), ((), ())),
        precision=lax.Precision.HIGHEST, preferred_element_type=jnp.float32)


_mm1 = pl.pallas_call(
    _mm1_body,
    grid=(N // _BM,),
    in_specs=[
        pl.BlockSpec((_BM, F), lambda i: (i, 0)),
        pl.BlockSpec((F, H), lambda i: (0, 0)),
    ],
    out_specs=pl.BlockSpec((_BM, H), lambda i: (i, 0)),
    out_shape=jax.ShapeDtypeStruct((N, H), jnp.float32),
)


_BN = 2000


def _scale_body(degp_ref, h1_ref, h1s_ref, dsi_ref, dso_ref):
    d = degp_ref[...]
    dso = lax.rsqrt(jnp.maximum(d[:, 0:1] + d[:, 2:3], 1.0))
    dsi = lax.rsqrt(jnp.maximum(d[:, 1:2] + d[:, 3:4], 1.0))
    h1s_ref[...] = h1_ref[...] * dso
    dsi_ref[...] = dsi
    dso_ref[...] = dso


_scale = pl.pallas_call(
    _scale_body,
    grid=(N // _BN,),
    in_specs=[
        pl.BlockSpec((_BN, 4), lambda i: (i, 0)),
        pl.BlockSpec((_BN, H), lambda i: (i, 0)),
    ],
    out_specs=[
        pl.BlockSpec((_BN, H), lambda i: (i, 0)),
        pl.BlockSpec((_BN, 1), lambda i: (i, 0)),
        pl.BlockSpec((_BN, 1), lambda i: (i, 0)),
    ],
    out_shape=[
        jax.ShapeDtypeStruct((N, H), jnp.float32),
        jax.ShapeDtypeStruct((N, 1), jnp.float32),
        jax.ShapeDtypeStruct((N, 1), jnp.float32),
    ],
)


def _post1_body(p_ref, dsi_ref, dso_ref, b1_ref, g_ref):
    q = (p_ref[0] + p_ref[1]) * dsi_ref[...]
    g_ref[...] = jnp.maximum(q + b1_ref[...], 0.0) * dso_ref[...]


_post1 = pl.pallas_call(
    _post1_body,
    grid=(N // _BN,),
    in_specs=[
        pl.BlockSpec((NC, _BN, H), lambda i: (0, i, 0)),
        pl.BlockSpec((_BN, 1), lambda i: (i, 0)),
        pl.BlockSpec((_BN, 1), lambda i: (i, 0)),
        pl.BlockSpec((1, H), lambda i: (0, 0)),
    ],
    out_specs=pl.BlockSpec((_BN, H), lambda i: (i, 0)),
    out_shape=jax.ShapeDtypeStruct((N, H), jnp.float32),
)


def _post2_body(p_ref, dsi_ref, w2_ref, b2_ref, o_ref):
    q = (p_ref[0] + p_ref[1]) * dsi_ref[...]
    o_ref[...] = lax.dot_general(
        q, w2_ref[...], (((1,), (0,)), ((), ())),
        precision=lax.Precision.HIGHEST,
        preferred_element_type=jnp.float32) + b2_ref[...]


_post2 = pl.pallas_call(
    _post2_body,
    grid=(N // _BN,),
    in_specs=[
        pl.BlockSpec((NC, _BN, H), lambda i: (0, i, 0)),
        pl.BlockSpec((_BN, 1), lambda i: (i, 0)),
        pl.BlockSpec((H, O), lambda i: (0, 0)),
        pl.BlockSpec((1, O), lambda i: (0, 0)),
    ],
    out_specs=pl.BlockSpec((_BN, O), lambda i: (i, 0)),
    out_shape=jax.ShapeDtypeStruct((N, O), jnp.float32),
)


def kernel(features, edge_index, W1, b1, W2, b2):
    pad = jnp.full((2, EPAD - E), SINK, jnp.int32)
    ei = jnp.concatenate([edge_index, pad], axis=1)
    ei = ei.reshape(2, ROWS, CH).transpose(1, 0, 2)

    degp = _deg(ei)
    degp4 = degp[:, :, :N].reshape(4, N).T
    h1 = _mm1(features, W1)
    h1s, dsi, dso = _scale(degp4, h1)
    h1s = jnp.pad(h1s, ((0, APAD - N), (0, 0)))
    p1 = _agg(h1s, ei)[:, :N]
    g = _post1(p1, dsi, dso, b1.reshape(1, H))
    g = jnp.pad(g, ((0, APAD - N), (0, 0)))
    p2 = _agg(g, ei)[:, :N]
    out = _post2(p2, dsi, W2, b2.reshape(1, O))
    return out


# speedup vs baseline: 15.3876x; 1.2745x over previous
import jax
import jax.numpy as jnp
from jax import lax
from jax.experimental import pallas as pl
from jax.experimental.pallas import tpu as pltpu
from jax.experimental.pallas import tpu_sc as plsc

N = 50000
E = 1600000
F = 1433
H = 16
O = 7

NC = 2
NS = 16
NW = NC * NS
CH = 128
ROWS = 12544
EPAD = ROWS * CH
WROWS = ROWS // NW
IBLK = 28
NOUTER = WROWS // IBLK
D = 4
SINK = N

NPAD = 51200
DSTRIPE = NPAD // NS
APAD = N + 48
ASTRIPE = APAD // NS
CLIP15 = N - 15 * ASTRIPE

_MESH = plsc.VectorSubcoreMesh(core_axis_name="c", subcore_axis_name="s")


def _deg_body(ei, out, acc_s, acc_d, zero_v, ones_v, iblk):
    c = lax.axis_index("c")
    s = lax.axis_index("s")
    w = c * NS + s

    zeros16 = jnp.zeros((16,), jnp.float32)

    def z_body(i, carry):
        zero_v[pl.ds(i * 16, 16)] = zeros16
        return carry

    lax.fori_loop(0, DSTRIPE // 16, z_body, None)
    for i in range(CH // 16):
        ones_v[pl.ds(i * 16, 16)] = jnp.ones((16,), jnp.float32)

    pltpu.sync_copy(zero_v, acc_s.at[pl.ds(s * DSTRIPE, DSTRIPE)])
    pltpu.sync_copy(zero_v, acc_d.at[pl.ds(s * DSTRIPE, DSTRIPE)])
    plsc.subcore_barrier()

    row0 = w * WROWS

    def outer(o, carry):
        pltpu.sync_copy(ei.at[pl.ds(row0 + o * IBLK, IBLK)], iblk)

        def inner(j, icarry):
            pltpu.sync_copy(ones_v, acc_s.at[iblk.at[j, 0]], add=True)
            pltpu.sync_copy(ones_v, acc_d.at[iblk.at[j, 1]], add=True)
            return icarry

        lax.fori_loop(0, IBLK, inner, None)
        return carry

    lax.fori_loop(0, NOUTER, outer, None)
    plsc.subcore_barrier()

    pltpu.sync_copy(acc_s.at[pl.ds(s * DSTRIPE, DSTRIPE)],
                    out.at[c, 0, pl.ds(s * DSTRIPE, DSTRIPE)])
    pltpu.sync_copy(acc_d.at[pl.ds(s * DSTRIPE, DSTRIPE)],
                    out.at[c, 1, pl.ds(s * DSTRIPE, DSTRIPE)])


_deg = pl.kernel(
    _deg_body,
    out_type=jax.ShapeDtypeStruct((NC, 2, NPAD), jnp.float32),
    mesh=_MESH,
    compiler_params=pltpu.CompilerParams(use_tc_tiling_on_sc=False),
    scratch_types=[
        pltpu.VMEM_SHARED((NPAD,), jnp.float32),
        pltpu.VMEM_SHARED((NPAD,), jnp.float32),
        pltpu.VMEM((DSTRIPE,), jnp.float32),
        pltpu.VMEM((CH,), jnp.float32),
        pltpu.VMEM((IBLK, 2, CH), jnp.int32),
    ],
)


def _agg_body(hs, ei, out, acc, zero_v, iblk, rows_v, gsem):
    c = lax.axis_index("c")
    s = lax.axis_index("s")
    w = c * NS + s

    zeros16 = jnp.zeros((16,), jnp.float32)

    def z_body(i, carry):
        zero_v[i, :] = zeros16
        return carry

    lax.fori_loop(0, ASTRIPE, z_body, None)
    pltpu.sync_copy(zero_v, acc.at[pl.ds(s * ASTRIPE, ASTRIPE), :])
    plsc.subcore_barrier()

    row0 = w * WROWS

    def outer(o, carry):
        pltpu.sync_copy(ei.at[pl.ds(row0 + o * IBLK, IBLK)], iblk)
        for k in range(D):
            pltpu.async_copy(hs.at[iblk.at[k, 0]], rows_v.at[k], gsem.at[k])

        def inner(j, icarry):
            b = lax.rem(j, D)
            pltpu.make_async_copy(hs.at[iblk.at[j, 0]], rows_v.at[b],
                                  gsem.at[b]).wait()
            pltpu.sync_copy(rows_v.at[b], acc.at[iblk.at[j, 1]], add=True)

            @pl.when(j < IBLK - D)
            def _():
                pltpu.async_copy(hs.at[iblk.at[j + D, 0]], rows_v.at[b],
                                 gsem.at[b])

            return icarry

        lax.fori_loop(0, IBLK, inner, None)
        return carry

    lax.fori_loop(0, NOUTER, outer, None)
    plsc.subcore_barrier()

    @pl.when(s < 15)
    def _():
        pltpu.sync_copy(acc.at[pl.ds(s * ASTRIPE, ASTRIPE), :],
                        out.at[c, pl.ds(s * ASTRIPE, ASTRIPE), :])

    @pl.when(s == 15)
    def _():
        pltpu.sync_copy(acc.at[pl.ds(15 * ASTRIPE, CLIP15), :],
                        out.at[c, pl.ds(15 * ASTRIPE, CLIP15), :])


_agg = pl.kernel(
    _agg_body,
    out_type=jax.ShapeDtypeStruct((NC, N, H), jnp.float32),
    mesh=_MESH,
    compiler_params=pltpu.CompilerParams(use_tc_tiling_on_sc=False),
    scratch_types=[
        pltpu.VMEM_SHARED((APAD, H), jnp.float32),
        pltpu.VMEM((ASTRIPE, H), jnp.float32),
        pltpu.VMEM((IBLK, 2, CH), jnp.int32),
        pltpu.VMEM((D, CH, H), jnp.float32),
        pltpu.SemaphoreType.DMA((D,)),
    ],
)


_BM = 1000


def _mm1_body(x_ref, w_ref, o_ref):
    o_ref[...] = lax.dot_general(
        x_ref[...], w_ref[...], (((1,), (0,)), ((), ())),
        preferred_element_type=jnp.float32)


_mm1 = pl.pallas_call(
    _mm1_body,
    grid=(N // _BM,),
    in_specs=[
        pl.BlockSpec((_BM, F), lambda i: (i, 0)),
        pl.BlockSpec((F, H), lambda i: (0, 0)),
    ],
    out_specs=pl.BlockSpec((_BM, H), lambda i: (i, 0)),
    out_shape=jax.ShapeDtypeStruct((N, H), jnp.float32),
)


_BN = 2000


def _scale_body(degp_ref, h1_ref, h1s_ref, dsi_ref, dso_ref):
    d = degp_ref[...]
    dso = lax.rsqrt(jnp.maximum(d[:, 0:1] + d[:, 2:3], 1.0))
    dsi = lax.rsqrt(jnp.maximum(d[:, 1:2] + d[:, 3:4], 1.0))
    h1s_ref[...] = h1_ref[...] * dso
    dsi_ref[...] = dsi
    dso_ref[...] = dso


_scale = pl.pallas_call(
    _scale_body,
    grid=(N // _BN,),
    in_specs=[
        pl.BlockSpec((_BN, 4), lambda i: (i, 0)),
        pl.BlockSpec((_BN, H), lambda i: (i, 0)),
    ],
    out_specs=[
        pl.BlockSpec((_BN, H), lambda i: (i, 0)),
        pl.BlockSpec((_BN, 1), lambda i: (i, 0)),
        pl.BlockSpec((_BN, 1), lambda i: (i, 0)),
    ],
    out_shape=[
        jax.ShapeDtypeStruct((APAD, H), jnp.float32),
        jax.ShapeDtypeStruct((N, 1), jnp.float32),
        jax.ShapeDtypeStruct((N, 1), jnp.float32),
    ],
)


def _post1_body(p_ref, dsi_ref, dso_ref, b1_ref, g_ref):
    q = (p_ref[0] + p_ref[1]) * dsi_ref[...]
    g_ref[...] = jnp.maximum(q + b1_ref[...], 0.0) * dso_ref[...]


_post1 = pl.pallas_call(
    _post1_body,
    grid=(N // _BN,),
    in_specs=[
        pl.BlockSpec((NC, _BN, H), lambda i: (0, i, 0)),
        pl.BlockSpec((_BN, 1), lambda i: (i, 0)),
        pl.BlockSpec((_BN, 1), lambda i: (i, 0)),
        pl.BlockSpec((1, H), lambda i: (0, 0)),
    ],
    out_specs=pl.BlockSpec((_BN, H), lambda i: (i, 0)),
    out_shape=jax.ShapeDtypeStruct((APAD, H), jnp.float32),
)


def _post2_body(p_ref, dsi_ref, w2_ref, b2_ref, o_ref):
    q = (p_ref[0] + p_ref[1]) * dsi_ref[...]
    o_ref[...] = lax.dot_general(
        q, w2_ref[...], (((1,), (0,)), ((), ())),
        preferred_element_type=jnp.float32) + b2_ref[...]


_post2 = pl.pallas_call(
    _post2_body,
    grid=(N // _BN,),
    in_specs=[
        pl.BlockSpec((NC, _BN, H), lambda i: (0, i, 0)),
        pl.BlockSpec((_BN, 1), lambda i: (i, 0)),
        pl.BlockSpec((H, O), lambda i: (0, 0)),
        pl.BlockSpec((1, O), lambda i: (0, 0)),
    ],
    out_specs=pl.BlockSpec((_BN, O), lambda i: (i, 0)),
    out_shape=jax.ShapeDtypeStruct((N, O), jnp.float32),
)


def kernel(features, edge_index, W1, b1, W2, b2):
    pad = jnp.full((2, EPAD - E), SINK, jnp.int32)
    ei = jnp.concatenate([edge_index, pad], axis=1)
    ei = ei.reshape(2, ROWS, CH).transpose(1, 0, 2)

    degp = _deg(ei)
    degp4 = degp[:, :, :N].reshape(4, N).T
    h1 = _mm1(features, W1)
    h1s, dsi, dso = _scale(degp4, h1)
    p1 = _agg(h1s, ei)
    g = _post1(p1, dsi, dso, b1.reshape(1, H))
    p2 = _agg(g, ei)
    out = _post2(p2, dsi, W2, b2.reshape(1, O))
    return out
